# baseline (device time: 466005 ns/iter reference)
import jax
import jax.numpy as jnp
from jax import lax
from jax.experimental import pallas as pl
from jax.experimental.pallas import tpu as pltpu

N_DEV = 8
S = 4096
D = 1024
BLK = 512
NB = S // BLK
U = S // (2 * N_DEV)
HS = S // 2
H_LOC = 8
DH = 128
SCALE = 0.08838834764831843
EPS = 1e-5
BF = jnp.bfloat16
F32 = jnp.float32


def _ln_mod(xb, s_ref, sh_ref):
    mu = jnp.mean(xb, axis=-1, keepdims=True)
    var = jnp.mean((xb - mu) ** 2, axis=-1, keepdims=True)
    xn = (xb - mu) * lax.rsqrt(var + EPS)
    return xn * (1.0 + s_ref[...]) + sh_ref[...]


def _qkv_body(x_ref, sa_ref, sha_ref, wq_ref, wk_ref, wv_ref,
              q_ref, k_ref, v_ref):
    xm = _ln_mod(x_ref[...], sa_ref, sha_ref).astype(BF)
    q_ref[...] = (jnp.dot(xm, wq_ref[...], preferred_element_type=F32)
                  * SCALE).astype(BF)
    k_ref[...] = jnp.dot(xm, wk_ref[...], preferred_element_type=F32).astype(BF)
    v_ref[...] = jnp.dot(xm, wv_ref[...], preferred_element_type=F32).astype(BF)


def _m8(v):
    return lax.rem(v + N_DEV, N_DEV)


def _top(c):
    return pl.ds(c * U, U)


def _bot(c):
    return pl.ds(HS + c * U, U)


def _neighbors():
    d = lax.axis_index("i")
    return d, _m8(d - 1), _m8(d + 1)


def _entry_barrier(left, right):
    barrier = pltpu.get_barrier_semaphore()
    for nbr in (left, right):
        pl.semaphore_signal(barrier, inc=1, device_id=(nbr,),
                            device_id_type=pl.DeviceIdType.MESH)
    pl.semaphore_wait(barrier, 2)


def _exit_barrier(left, right):
    def _second(second_barrier):
        for nbr in (left, right):
            pl.semaphore_signal(second_barrier, inc=1, device_id=(nbr,),
                                device_id_type=pl.DeviceIdType.MESH)
        pl.semaphore_wait(second_barrier, 2)

    pl.run_scoped(_second, second_barrier=pltpu.SemaphoreType.REGULAR)


def _ring_ops(send_r, recv_r, send_l, recv_l, left, right):
    def rdma_pair(src_r, dst_r, src_l, dst_l, step):
        rr = pltpu.make_async_remote_copy(
            src_ref=src_r, dst_ref=dst_r,
            send_sem=send_r.at[step], recv_sem=recv_r.at[step],
            device_id=(right,), device_id_type=pl.DeviceIdType.MESH)
        rl = pltpu.make_async_remote_copy(
            src_ref=src_l, dst_ref=dst_l,
            send_sem=send_l.at[step], recv_sem=recv_l.at[step],
            device_id=(left,), device_id_type=pl.DeviceIdType.MESH)
        rr.start()
        rl.start()
        return rr, rl

    def wait(pair):
        pair[0].wait()
        pair[1].wait()

    return rdma_pair, wait


def _rs_jit(compute, p_src, rdma_pair, wait, comm_r, comm_l, d):
    compute(_top(_m8(d)), p_src(0)[0])
    compute(_bot(_m8(d)), p_src(0)[1])
    pair = rdma_pair(p_src(0)[0], comm_r.at[0], p_src(0)[1], comm_l.at[0], 0)
    for s in range(1, N_DEV - 1):
        slot = s % 2
        compute(_top(_m8(d - s)), p_src(slot)[0])
        compute(_bot(_m8(d + s)), p_src(slot)[1])
        wait(pair)
        comm_r[s - 1] = comm_r[s - 1] + p_src(slot)[0][...]
        comm_l[s - 1] = comm_l[s - 1] + p_src(slot)[1][...]
        pair = rdma_pair(comm_r.at[s - 1], comm_r.at[s],
                         comm_l.at[s - 1], comm_l.at[s], s)
    slot = (N_DEV - 1) % 2
    compute(_top(_m8(d + 1)), p_src(slot)[0])
    compute(_bot(_m8(d - 1)), p_src(slot)[1])
    wait(pair)
    tot_r = comm_r[N_DEV - 2].astype(F32) + p_src(slot)[0][...].astype(F32)
    tot_l = comm_l[N_DEV - 2].astype(F32) + p_src(slot)[1][...].astype(F32)
    return tot_r, tot_l


def _ag(buf, rdma_pair, wait, d):
    own_t = _m8(d + 1)
    own_b = _m8(d - 1)
    pair = rdma_pair(buf.at[_top(own_t)], buf.at[_top(own_t)],
                     buf.at[_bot(own_b)], buf.at[_bot(own_b)], 0)
    for t in range(1, N_DEV - 1):
        wait(pair)
        ct = _m8(d + 1 - t)
        cb = _m8(d - 1 + t)
        pair = rdma_pair(buf.at[_top(ct)], buf.at[_top(ct)],
                         buf.at[_bot(cb)], buf.at[_bot(cb)], t)
    wait(pair)


def _attn_body(q_ref, k_ref, v_ref, wo_ref, x0_ref, ga_ref, x1_ref,
               pbuf_r, pbuf_l, attn_buf, comm_r, comm_l,
               send_r, recv_r, send_l, recv_l):
    d, left, right = _neighbors()
    _entry_barrier(left, right)
    rdma_pair, wait = _ring_ops(send_r, recv_r, send_l, recv_l, left, right)

    def attn_proj(rows_slc, dst):
        def head_body(h, _):
            hc = pl.ds(h * DH, DH)
            qh = q_ref[rows_slc, hc]
            s_ = lax.dot_general(qh, k_ref[:, hc], (((1,), (1,)), ((), ())),
                                 preferred_element_type=F32)
            p_ = jnp.exp(s_.astype(BF))
            l_ = jnp.sum(p_, axis=-1, keepdims=True, dtype=F32)
            o_ = jnp.dot(p_, v_ref[:, hc], preferred_element_type=F32)
            attn_buf[:, hc] = (o_ / l_).astype(BF)
            return 0
        lax.fori_loop(0, H_LOC, head_body, 0)
        dst[...] = jnp.dot(attn_buf[...], wo_ref[...],
                           preferred_element_type=F32).astype(BF)

    def p_src(slot):
        return pbuf_r.at[slot], pbuf_l.at[slot]

    tot_r, tot_l = _rs_jit(attn_proj, p_src, rdma_pair, wait,
                           comm_r, comm_l, d)

    ga = ga_ref[...]
    own_t = _m8(d + 1)
    own_b = _m8(d - 1)
    x1_ref[_top(own_t), :] = (x0_ref[_top(own_t), :].astype(F32)
                              + ga * tot_r).astype(BF)
    x1_ref[_bot(own_b), :] = (x0_ref[_bot(own_b), :].astype(F32)
                              + ga * tot_l).astype(BF)

    _ag(x1_ref, rdma_pair, wait, d)
    _exit_barrier(left, right)


def _ffn_body(x1_ref, sm_ref, shm_ref, gm_ref, w1_ref, w2_ref, out_ref,
              pbuf_r, pbuf_l, comm_r, comm_l,
              send_r, recv_r, send_l, recv_l):
    d, left, right = _neighbors()
    _entry_barrier(left, right)
    rdma_pair, wait = _ring_ops(send_r, recv_r, send_l, recv_l, left, right)

    def ffn(rows_slc, dst):
        xb = x1_ref[rows_slc, :].astype(F32)
        xm = _ln_mod(xb, sm_ref, shm_ref).astype(BF)
        h_ = jnp.dot(xm, w1_ref[...], preferred_element_type=F32)
        h_ = h_ * jax.nn.sigmoid(h_)
        dst[...] = jnp.dot(h_.astype(BF), w2_ref[...],
                           preferred_element_type=F32).astype(BF)

    def p_src(slot):
        return pbuf_r.at[slot], pbuf_l.at[slot]

    tot_r, tot_l = _rs_jit(ffn, p_src, rdma_pair, wait, comm_r, comm_l, d)

    gm = gm_ref[...]
    own_t = _m8(d + 1)
    own_b = _m8(d - 1)
    out_ref[_top(own_t), :] = (x1_ref[_top(own_t), :].astype(F32)
                               + gm * tot_r).astype(BF)
    out_ref[_bot(own_b), :] = (x1_ref[_bot(own_b), :].astype(F32)
                               + gm * tot_l).astype(BF)

    _ag(out_ref, rdma_pair, wait, d)
    _exit_barrier(left, right)


def kernel(x, Wq, Wk, Wv, Wo, t_emb, W_mod, W_ff1, W_ff2):
    x0 = x[0]
    mod = t_emb @ W_mod
    sa, sha, ga, sm, shm, gm = jnp.split(mod, 6, axis=-1)

    Wq_b = Wq.astype(BF)
    Wk_b = Wk.astype(BF)
    Wv_b = Wv.astype(BF)
    Wo_b = Wo.astype(BF)
    W1_b = W_ff1.astype(BF)
    W2_b = W_ff2.astype(BF)

    row = pl.BlockSpec((BLK, D), lambda i: (i, 0))
    vec = pl.BlockSpec((1, D), lambda i: (0, 0))
    wfull = pl.BlockSpec((D, D), lambda i: (0, 0))

    q, k, v = pl.pallas_call(
        _qkv_body,
        grid=(NB,),
        in_specs=[row, vec, vec, wfull, wfull, wfull],
        out_specs=[row, row, row],
        out_shape=[jax.ShapeDtypeStruct((S, D), BF)] * 3,
    )(x0, sa, sha, Wq_b, Wk_b, Wv_b)

    sems = [pltpu.SemaphoreType.DMA((N_DEV - 1,))] * 4

    x1 = pl.pallas_call(
        _attn_body,
        out_shape=jax.ShapeDtypeStruct((S, D), BF),
        in_specs=[pl.BlockSpec(memory_space=pltpu.VMEM)] * 6,
        out_specs=pl.BlockSpec(memory_space=pltpu.VMEM),
        scratch_shapes=[
            pltpu.VMEM((2, U, D), BF),
            pltpu.VMEM((2, U, D), BF),
            pltpu.VMEM((U, D), BF),
            pltpu.VMEM((N_DEV - 1, U, D), BF),
            pltpu.VMEM((N_DEV - 1, U, D), BF),
        ] + sems,
        compiler_params=pltpu.CompilerParams(
            collective_id=0, vmem_limit_bytes=80 * 1024 * 1024),
    )(q, k, v, Wo_b, x0.astype(BF), ga)

    out = pl.pallas_call(
        _ffn_body,
        out_shape=jax.ShapeDtypeStruct((S, D), BF),
        in_specs=[pl.BlockSpec(memory_space=pltpu.VMEM)] * 6,
        out_specs=pl.BlockSpec(memory_space=pltpu.VMEM),
        scratch_shapes=[
            pltpu.VMEM((2, U, D), BF),
            pltpu.VMEM((2, U, D), BF),
            pltpu.VMEM((N_DEV - 1, U, D), BF),
            pltpu.VMEM((N_DEV - 1, U, D), BF),
        ] + sems,
        compiler_params=pltpu.CompilerParams(
            collective_id=1, vmem_limit_bytes=80 * 1024 * 1024),
    )(x1, sm, shm, gm, W1_b, W2_b)

    return out.astype(F32)[None]
